# baseline (speedup 1.0000x reference)
import functools
import jax
import jax.numpy as jnp
from jax import lax
from jax.experimental import pallas as pl
from jax.experimental.pallas import tpu as pltpu
from jax.experimental.pallas import tpu_sc as plsc

_T, _N, _D = 4, 50000, 128
_ROWS = _T * _N
_R = 320
_NCHUNKS = _ROWS // _R
_NW = 32


def _sc_body(nd_hbm, obs_hbm, pos_hbm, neg_hbm, out_hbm,
             spkp, spkn, buf0, buf1, buf2, ob0, ob1, ob2,
             si0, si1, si2, so0, so1, so2, sspk):
    bufs = (buf0, buf1, buf2)
    obsbs = (ob0, ob1, ob2)
    sins = (si0, si1, si2)
    souts = (so0, so1, so2)
    w = lax.axis_index("s") * 2 + lax.axis_index("c")
    pltpu.async_copy(pos_hbm, spkp, sspk)
    pltpu.async_copy(neg_hbm, spkn, sspk)
    n_mine = (_NCHUNKS - w + _NW - 1) // _NW

    def chunk_base(j):
        return (w + j * _NW) * _R

    def start_in(b, j):
        base = chunk_base(j)
        pltpu.async_copy(nd_hbm.at[pl.ds(base, _R)], bufs[b], sins[b])
        pltpu.async_copy(obs_hbm.at[pl.ds(base, _R)], obsbs[b], sins[b])

    def wait_in(b, j):
        base = chunk_base(j)
        pltpu.make_async_copy(nd_hbm.at[pl.ds(base, _R)], bufs[b],
                              sins[b]).wait()
        pltpu.make_async_copy(obs_hbm.at[pl.ds(base, _R)], obsbs[b],
                              sins[b]).wait()

    def wait_out(b, j):
        pltpu.make_async_copy(bufs[b], out_hbm.at[pl.ds(chunk_base(j), _R)],
                              souts[b]).wait()

    def compute_store(b, j):
        buf, obsb = bufs[b], obsbs[b]

        def group_body(g, c2):
            ov = obsb[pl.ds(g * 16, 16)]
            for k in range(16):
                r = g * 16 + k
                o = ov[k]
                po = (o == 1).astype(jnp.float32)
                ng = (o == -1).astype(jnp.float32)
                for s in range(8):
                    sl = pl.ds(s * 16, 16)
                    buf[r, sl] = buf[r, sl] + po * pseg[s] + ng * nseg[s]
            return c2

        lax.fori_loop(0, _R // 16, group_body, 0)
        pltpu.async_copy(buf, out_hbm.at[pl.ds(chunk_base(j), _R)], souts[b])

    @pl.when(n_mine > 0)
    def _():
        start_in(0, 0)

    @pl.when(n_mine > 1)
    def _():
        start_in(1, 1)

    pltpu.make_async_copy(pos_hbm, spkp, sspk).wait()
    pltpu.make_async_copy(neg_hbm, spkn, sspk).wait()
    pseg = [spkp[pl.ds(s * 16, 16)] for s in range(8)]
    nseg = [spkn[pl.ds(s * 16, 16)] for s in range(8)]

    def iter_body(p, carry):
        for b in range(3):
            j = 3 * p + b

            @pl.when(j < n_mine)
            def _():
                wait_in(b, j)
                compute_store(b, j)
                jn = j + 2
                bn = (b + 2) % 3

                @pl.when(jn < n_mine)
                def _():
                    @pl.when(j >= 1)
                    def _():
                        wait_out(bn, j - 1)

                    start_in(bn, jn)

        return carry

    lax.fori_loop(0, (n_mine + 2) // 3, iter_body, 0)

    for b in range(3):
        @pl.when(n_mine > b)
        def _(b=b):
            last = n_mine - 1
            off = lax.rem(last - b + 3, 3)
            wait_out(b, last - off)


def kernel(node_data, observations, pos_test_spike, neg_test_spike):
    nd = node_data.reshape(_ROWS, _D)
    obs = observations.reshape(_ROWS).astype(jnp.int32)

    sc = functools.partial(
        pl.kernel,
        mesh=plsc.VectorSubcoreMesh(core_axis_name="c", subcore_axis_name="s"),
        out_type=jax.ShapeDtypeStruct((_ROWS, _D), jnp.float32),
        scratch_types=(
            [pltpu.VMEM((_D,), jnp.float32) for _ in range(2)]
            + [pltpu.VMEM((_R, _D), jnp.float32) for _ in range(3)]
            + [pltpu.VMEM((_R,), jnp.int32) for _ in range(3)]
            + [pltpu.SemaphoreType.DMA for _ in range(7)]
        ),
    )(_sc_body)
    out = sc(nd, obs, pos_test_spike, neg_test_spike)
    return out.reshape(_T, _N, _D)
